# baseline (speedup 1.0000x reference)
import jax
import jax.numpy as jnp
from jax.experimental import pallas as pl
from jax.experimental.pallas import tpu as pltpu

_E = 8
_TOPK = 2
_CHUNK = 1024


def _routing(logits):
    m1 = jnp.max(logits, axis=-1, keepdims=True)
    i1 = jnp.argmax(logits, axis=-1)
    iota = jax.lax.broadcasted_iota(jnp.int32, logits.shape, 1)
    masked = jnp.where(iota == i1[:, None], -jnp.inf, logits)
    m2 = jnp.max(masked, axis=-1, keepdims=True)
    i2 = jnp.argmax(masked, axis=-1)
    w1 = 1.0 / (1.0 + jnp.exp(m2 - m1))
    vals = jnp.concatenate([w1, 1.0 - w1], axis=1)
    idx = jnp.concatenate([i1[:, None], i2[:, None]], axis=1)
    return vals, idx


def _pipeline_body(x_ref, w_ref, b_ref, logits_ref, vals_ref, idx_ref):
    logits = jax.lax.dot_general(
        x_ref[...], w_ref[...], (((1,), (1,)), ((), ())),
        preferred_element_type=jnp.float32,
    ) + b_ref[...]
    logits_ref[...] = logits
    vals, idx = _routing(logits)
    vals_ref[...] = vals
    idx_ref[...] = idx


def kernel(hidden_states, W, b):
    orig_shape = hidden_states.shape
    x = hidden_states.reshape(-1, orig_shape[-1])
    n_tokens, hidden = x.shape
    n_chunks = n_tokens // _CHUNK
    mesh = pltpu.create_tensorcore_mesh("core")

    def run(refs):
        x_ref, w_ref, b_ref, logits_ref, vals_ref, idx_ref = refs

        @pl.core_map(mesh)
        def _per_core():
            pipeline = pltpu.emit_pipeline(
                _pipeline_body,
                grid=(n_chunks,),
                in_specs=[
                    pl.BlockSpec((_CHUNK, hidden), lambda i: (i, 0)),
                    pl.BlockSpec((_E, hidden), lambda i: (0, 0)),
                    pl.BlockSpec((1, _E), lambda i: (0, 0)),
                ],
                out_specs=[
                    pl.BlockSpec((_CHUNK, _E), lambda i: (i, 0)),
                    pl.BlockSpec((_CHUNK, _TOPK), lambda i: (i, 0)),
                    pl.BlockSpec((_CHUNK, _TOPK), lambda i: (i, 0)),
                ],
                core_axis_name="core",
            )
            pipeline(x_ref, w_ref, b_ref, logits_ref, vals_ref, idx_ref)

    _, _, _, logits, vals, idx = pl.run_state(run)(
        (
            x,
            W,
            b.reshape(1, _E),
            jnp.zeros((n_tokens, _E), jnp.float32),
            jnp.zeros((n_tokens, _TOPK), jnp.float32),
            jnp.zeros((n_tokens, _TOPK), jnp.int32),
        )
    )
    return (logits, vals, idx)
